# baseline (speedup 1.0000x reference)
import functools

import jax
import jax.numpy as jnp
from jax import lax
from jax.experimental import pallas as pl
from jax.experimental.pallas import tpu as pltpu
from jax.experimental.pallas import tpu_sc as plsc

_NUM_CORES = 2
_NUM_SUBCORES = 16
_LANES = 16
_CHUNK = 128


def _embedding_bag_sc(indices_3d, bagmap_3d, table, batch, hist, dim):
    n_workers = _NUM_CORES * _NUM_SUBCORES
    bags_per_w = batch // n_workers
    bags_per_core = bags_per_w * _NUM_SUBCORES
    rows_per_w = bags_per_w * hist
    n_chunks = rows_per_w // _CHUNK

    mesh = plsc.VectorSubcoreMesh(core_axis_name="c", subcore_axis_name="s")

    @functools.partial(
        pl.kernel,
        out_type=jax.ShapeDtypeStruct((batch, dim), jnp.float32),
        mesh=mesh,
        scratch_types=[
            pltpu.VMEM((n_chunks, _CHUNK), jnp.int32),
            pltpu.VMEM((n_chunks, _CHUNK), jnp.int32),
            pltpu.VMEM((2, _CHUNK, dim), jnp.float32),
            pltpu.VMEM_SHARED((bags_per_core, dim), jnp.float32),
            pltpu.SemaphoreType.DMA,
            pltpu.SemaphoreType.DMA,
        ],
        compiler_params=pltpu.CompilerParams(use_tc_tiling_on_sc=False),
    )
    def ker(idx_hbm, bag_hbm, table_hbm, out_hbm, idx_v, bag_v, rows_v,
            acc_sh, sem0, sem1):
        c = lax.axis_index("c")
        s = lax.axis_index("s")
        w = c * _NUM_SUBCORES + s

        pltpu.sync_copy(idx_hbm.at[w], idx_v)
        pltpu.sync_copy(bag_hbm.at[w], bag_v)

        @pl.loop(0, _CHUNK)
        def _(b):
            for d in range(0, dim, _LANES):
                rows_v[0, b, pl.ds(d, _LANES)] = jnp.zeros((_LANES,),
                                                           jnp.float32)
        for z in range(0, bags_per_w, _CHUNK):
            pltpu.sync_copy(rows_v.at[0],
                            acc_sh.at[pl.ds(s * bags_per_w + z, _CHUNK)])

        sems = (sem0, sem1)
        descs = [None, None]
        for g in range(n_chunks):
            b = g % 2
            descs[b] = pltpu.async_copy(
                table_hbm.at[idx_v.at[g]], rows_v.at[b], sems[b])
            if g >= 1:
                pb = (g - 1) % 2
                descs[pb].wait()
                pltpu.sync_copy(rows_v.at[pb], acc_sh.at[bag_v.at[g - 1]],
                                add=True)
        lb = (n_chunks - 1) % 2
        descs[lb].wait()
        pltpu.sync_copy(rows_v.at[lb], acc_sh.at[bag_v.at[n_chunks - 1]],
                        add=True)

        pltpu.sync_copy(acc_sh.at[pl.ds(s * bags_per_w, bags_per_w)],
                        out_hbm.at[pl.ds(w * bags_per_w, bags_per_w)])

    return ker(indices_3d, bagmap_3d, table)


def kernel(indices, table):
    batch, hist = indices.shape
    _, dim = table.shape
    n_workers = _NUM_CORES * _NUM_SUBCORES
    rows_total = batch * hist
    assert rows_total % (n_workers * _CHUNK) == 0
    assert batch % n_workers == 0 and dim % _LANES == 0

    n_chunks = rows_total // (n_workers * _CHUNK)
    bags_per_w = batch // n_workers

    flat = indices.reshape(rows_total).astype(jnp.int32)
    idx_3d = flat.reshape(n_workers, n_chunks, _CHUNK)
    bags_per_core = bags_per_w * _NUM_SUBCORES
    bagmap = (
        (jnp.arange(rows_total, dtype=jnp.int32) // hist) % bags_per_core
    ).reshape(n_workers, n_chunks, _CHUNK)
    return _embedding_bag_sc(idx_3d, bagmap, table.astype(jnp.float32),
                             batch, hist, dim)


# speedup vs baseline: 1.0021x; 1.0021x over previous
import functools

import jax
import jax.numpy as jnp
from jax import lax
from jax.experimental import pallas as pl
from jax.experimental.pallas import tpu as pltpu
from jax.experimental.pallas import tpu_sc as plsc

_NUM_CORES = 2
_NUM_SUBCORES = 16
_LANES = 16
_CHUNK = 128


def _embedding_bag_sc(indices_t, table, batch, hist, dim):
    n_workers = _NUM_CORES * _NUM_SUBCORES
    bags_per_w = batch // n_workers
    bags_per_core = bags_per_w * _NUM_SUBCORES

    mesh = plsc.VectorSubcoreMesh(core_axis_name="c", subcore_axis_name="s")

    @functools.partial(
        pl.kernel,
        out_type=jax.ShapeDtypeStruct((batch, dim), jnp.float32),
        mesh=mesh,
        scratch_types=[
            pltpu.VMEM((hist, _CHUNK), jnp.int32),
            pltpu.VMEM((_CHUNK,), jnp.int32),
            pltpu.VMEM((2, _CHUNK, dim), jnp.float32),
            pltpu.VMEM_SHARED((bags_per_core, dim), jnp.float32),
            pltpu.SemaphoreType.DMA,
            pltpu.SemaphoreType.DMA,
        ],
        compiler_params=pltpu.CompilerParams(use_tc_tiling_on_sc=False),
    )
    def ker(idx_hbm, table_hbm, out_hbm, idx_v, bag_v, rows_v, acc_sh,
            sem0, sem1):
        c = lax.axis_index("c")
        s = lax.axis_index("s")
        w = c * _NUM_SUBCORES + s
        b0 = w * bags_per_w

        pltpu.sync_copy(idx_hbm.at[:, pl.ds(b0, bags_per_w)], idx_v)

        sbase = s * bags_per_w
        for j in range(0, _CHUNK, _LANES):
            bag_v[pl.ds(j, _LANES)] = lax.iota(jnp.int32, _LANES) + (
                sbase + j)

        @pl.loop(0, _CHUNK)
        def _(b):
            for d in range(0, dim, _LANES):
                rows_v[0, b, pl.ds(d, _LANES)] = jnp.zeros((_LANES,),
                                                           jnp.float32)
        for z in range(0, bags_per_w, _CHUNK):
            pltpu.sync_copy(rows_v.at[0],
                            acc_sh.at[pl.ds(sbase + z, _CHUNK)])

        sems = (sem0, sem1)
        descs = [None, None]
        for l in range(hist):
            b = l % 2
            descs[b] = pltpu.async_copy(
                table_hbm.at[idx_v.at[l]], rows_v.at[b], sems[b])
            if l >= 1:
                pb = (l - 1) % 2
                descs[pb].wait()
                pltpu.sync_copy(rows_v.at[pb], acc_sh.at[bag_v], add=True)
        lb = (hist - 1) % 2
        descs[lb].wait()
        pltpu.sync_copy(rows_v.at[lb], acc_sh.at[bag_v], add=True)

        pltpu.sync_copy(acc_sh.at[pl.ds(sbase, bags_per_w)],
                        out_hbm.at[pl.ds(b0, bags_per_w)])

    return ker(indices_t, table)


def kernel(indices, table):
    batch, hist = indices.shape
    _, dim = table.shape
    n_workers = _NUM_CORES * _NUM_SUBCORES
    assert batch % (n_workers * _CHUNK) == 0 and dim % _LANES == 0
    assert batch // n_workers == _CHUNK

    indices_t = indices.T.astype(jnp.int32)
    return _embedding_bag_sc(indices_t, table.astype(jnp.float32),
                             batch, hist, dim)
